# baseline (speedup 1.0000x reference)
import functools

import jax
import jax.numpy as jnp
from jax import lax
from jax.experimental import pallas as pl
from jax.experimental.pallas import tpu as pltpu
from jax.experimental.pallas import tpu_sc as plsc

EMBED_DIM = 64
SEQ = 200
LANES = 16

NUM_CORES = 2
NUM_SUBCORES = 16
NUM_WORKERS = NUM_CORES * NUM_SUBCORES

CHUNK = 128
NBUF = 2
NST = 3
GROUP = 6


def _pe_table():
    pos = jnp.arange(SEQ, dtype=jnp.float32)[:, None]
    i = jnp.arange(0, EMBED_DIM, 2, dtype=jnp.float32)
    div = jnp.exp(-(jnp.log(10000.0) * i / EMBED_DIM))
    pe = jnp.zeros((SEQ, EMBED_DIM), dtype=jnp.float32)
    pe = pe.at[:, 0::2].set(jnp.sin(pos * div))
    pe = pe.at[:, 1::2].set(jnp.cos(pos * div))
    return pe


def _make_kernel(batch, seq):
    total = batch * seq
    per_w = total // NUM_WORKERS
    steps = per_w // CHUNK
    assert steps == SEQ and seq == SEQ

    mesh = plsc.VectorSubcoreMesh(
        core_axis_name="c", subcore_axis_name="s",
        num_cores=NUM_CORES, num_subcores=NUM_SUBCORES)

    @functools.partial(
        pl.kernel,
        out_type=jax.ShapeDtypeStruct((total, EMBED_DIM), jnp.float32),
        mesh=mesh,
        compiler_params=pltpu.CompilerParams(
            use_tc_tiling_on_sc=True, needs_layout_passes=False),
        scratch_types=[
            pltpu.VMEM((steps, CHUNK), jnp.int32),
            pltpu.VMEM((NBUF, CHUNK), jnp.int32),
            pltpu.VMEM((NBUF, CHUNK, 128), jnp.float32),
            pltpu.VMEM((NST, CHUNK, EMBED_DIM), jnp.float32),
        ]
        + [pltpu.SemaphoreType.DMA] * (NBUF + 2 * NST),
    )
    def k(x_hbm, tbl_hbm, pe_hbm, out_hbm, idx_v, p_v, gath_v, out_v,
          *sems):
        sem_g = sems[:NBUF]
        sem_s = sems[NBUF:NBUF + NST]
        sem_p = sems[NBUF + NST:]
        wid = lax.axis_index("s") * NUM_CORES + lax.axis_index("c")
        base = wid * per_w
        pltpu.sync_copy(x_hbm.at[wid], idx_v)

        def p0_of(s):
            return lax.rem(s * CHUNK, seq)

        def launch_gather(b, s):
            for g in range(CHUNK // LANES):
                sl = pl.ds(g * LANES, LANES)
                p_v[b, sl] = lax.shift_right_logical(idx_v[s, sl], 1)
            pltpu.async_copy(tbl_hbm.at[p_v.at[b]], gath_v.at[b], sem_g[b])

        def wait_gather(b):
            pltpu.make_async_copy(
                tbl_hbm.at[pl.ds(0, CHUNK)], gath_v.at[b], sem_g[b]).wait()

        def launch_premix(t, s):
            pltpu.async_copy(
                pe_hbm.at[pl.ds(p0_of(s), CHUNK)], out_v.at[t], sem_p[t])

        def wait_premix(t):
            pltpu.make_async_copy(
                pe_hbm.at[pl.ds(0, CHUNK)], out_v.at[t], sem_p[t]).wait()

        def launch_scatter(t, s):
            pltpu.async_copy(
                out_v.at[t], out_hbm.at[pl.ds(base + s * CHUNK, CHUNK)],
                sem_s[t])

        def wait_scatter(t):
            pltpu.make_async_copy(
                out_v.at[t], out_hbm.at[pl.ds(0, CHUNK)], sem_s[t]).wait()

        def accumulate(b, t, s):
            def g_body(g, _):
                idxs = idx_v[s, pl.ds(g * LANES, LANES)]
                hv = lax.shift_left(lax.bitwise_and(idxs, 1), 6)
                for r16 in range(LANES):
                    h = hv[r16]
                    rloc = g * LANES + r16
                    for c in range(EMBED_DIM // LANES):
                        plsc.addupdate(
                            out_v.at[t, rloc, pl.ds(c * LANES, LANES)],
                            gath_v[b, rloc, pl.ds(h + c * LANES, LANES)])
                return 0

            lax.fori_loop(0, CHUNK // LANES, g_body, 0)

        def step_body(s, b, t, *, first, premix_next=True, gather_next=True):
            wait_gather(b)
            wait_premix(t)
            accumulate(b, t, s)
            launch_scatter(t, s)
            t1 = (t + 1) % NST
            if not first:
                wait_scatter(t1)
            if premix_next:
                launch_premix(t1, s + 1)
            if gather_next:
                launch_gather(b, s + 2)

        launch_premix(0, 0)
        launch_gather(0, 0)
        launch_gather(1, 1)

        for ss in range(2):
            step_body(ss, ss % NBUF, ss % NST, first=True)

        groups = (steps - 2 - GROUP) // GROUP
        assert steps == 2 + GROUP * groups + GROUP

        def group_body(gi, _):
            s0 = 2 + gi * GROUP
            for off in range(GROUP):
                s = s0 + off
                step_body(s, (2 + off) % NBUF, (2 + off) % NST, first=False)
            return 0

        lax.fori_loop(0, groups, group_body, 0)

        for off in range(GROUP):
            ss = steps - GROUP + off
            step_body(ss, ss % NBUF, ss % NST, first=False,
                      premix_next=ss + 1 < steps, gather_next=ss + 2 < steps)

        wait_scatter((steps - 2) % NST)
        wait_scatter((steps - 1) % NST)

    return k


def kernel(x, token_embedding_weight):
    batch, seq = x.shape
    total = batch * seq
    xi = x.astype(jnp.int32).reshape(NUM_WORKERS, total // (NUM_WORKERS * CHUNK), CHUNK)
    w = token_embedding_weight
    tbl = jnp.concatenate([w[0::2], w[1::2]], axis=1)
    pe1 = _pe_table()
    pe = jnp.concatenate([pe1, pe1], axis=0)
    k = _make_kernel(batch, seq)
    out = k(xi, tbl, pe)
    return out.reshape(batch, seq, EMBED_DIM)


# speedup vs baseline: 3.9575x; 3.9575x over previous
import functools

import jax
import jax.numpy as jnp
from jax import lax
from jax.experimental import pallas as pl
from jax.experimental.pallas import tpu as pltpu
from jax.experimental.pallas import tpu_sc as plsc

EMBED_DIM = 64
SEQ = 200
LANES = 16
VOCAB = 1000000

NUM_CORES = 2
NUM_SUBCORES = 16
NUM_WORKERS = NUM_CORES * NUM_SUBCORES

CHUNK = 128
NBUF = 6

TBLK = 128
NFULL = VOCAB // TBLK
NTAIL = VOCAB - NFULL * TBLK


def _pe_table():
    pos = jnp.arange(SEQ, dtype=jnp.float32)[:, None]
    i = jnp.arange(0, EMBED_DIM, 2, dtype=jnp.float32)
    div = jnp.exp(-(jnp.log(10000.0) * i / EMBED_DIM))
    pe = jnp.zeros((SEQ, EMBED_DIM), dtype=jnp.float32)
    pe = pe.at[:, 0::2].set(jnp.sin(pos * div))
    pe = pe.at[:, 1::2].set(jnp.cos(pos * div))
    return pe


def _make_detile_kernel():
    mesh = plsc.VectorSubcoreMesh(
        core_axis_name="c", subcore_axis_name="s",
        num_cores=NUM_CORES, num_subcores=NUM_SUBCORES)
    blocks_ceil = (NFULL + NUM_WORKERS - 1) // NUM_WORKERS

    @functools.partial(
        pl.kernel,
        out_type=jax.ShapeDtypeStruct((VOCAB * EMBED_DIM,), jnp.float32),
        mesh=mesh,
        compiler_params=pltpu.CompilerParams(
            use_tc_tiling_on_sc=True, needs_layout_passes=False),
        scratch_types=[
            pltpu.VMEM((2, EMBED_DIM, TBLK), jnp.float32),
            pltpu.VMEM((2, TBLK * EMBED_DIM), jnp.float32),
            pltpu.VMEM((NTAIL, EMBED_DIM), jnp.float32),
        ]
        + [pltpu.SemaphoreType.DMA] * 4,
    )
    def k0(wt_hbm, wtail_hbm, out_hbm, in_v, tr_v, tail_v, *sems):
        sem_i = sems[:2]
        sem_o = sems[2:]
        wid = lax.axis_index("s") * NUM_CORES + lax.axis_index("c")
        nlast = jnp.where(wid < NFULL % NUM_WORKERS,
                          blocks_ceil - 1, blocks_ceil - 2)

        def blk(jj):
            return wid + NUM_WORKERS * jnp.minimum(jj, nlast)

        def launch_in(b, jj):
            pltpu.async_copy(
                wt_hbm.at[:, pl.ds(blk(jj) * TBLK, TBLK)], in_v.at[b],
                sem_i[b])

        def wait_in(b):
            pltpu.make_async_copy(
                wt_hbm.at[:, pl.ds(0, TBLK)], in_v.at[b], sem_i[b]).wait()

        def launch_out(b, jj):
            pltpu.async_copy(
                tr_v.at[b], out_hbm.at[pl.ds(blk(jj) * (TBLK * EMBED_DIM),
                                             TBLK * EMBED_DIM)], sem_o[b])

        def wait_out(b):
            pltpu.make_async_copy(
                tr_v.at[b],
                out_hbm.at[pl.ds(0, TBLK * EMBED_DIM)], sem_o[b]).wait()

        iota = lax.iota(jnp.int32, LANES)
        dcol = [iota + c * LANES for c in range(EMBED_DIM // LANES)]

        def transpose(b):
            def row_body(i, _):
                ibc = jnp.full((LANES,), 0, jnp.int32) + i
                for c in range(EMBED_DIM // LANES):
                    vals = plsc.load_gather(in_v.at[b], [dcol[c], ibc])
                    tr_v[b, pl.ds(i * EMBED_DIM + c * LANES, LANES)] = vals
                return 0

            lax.fori_loop(0, TBLK, row_body, 0, unroll=2)

        def jj_step(jj, b, *, first, launch_next):
            wait_in(b)
            if not first:
                wait_out(b)
            transpose(b)
            launch_out(b, jj)
            if launch_next:
                launch_in(b, jj + 2)

        launch_in(0, 0)
        launch_in(1, 1)

        for jj in range(2):
            jj_step(jj, jj % 2, first=True, launch_next=True)

        pairs = (blocks_ceil - 2 - 3) // 2
        assert blocks_ceil == 2 + 2 * pairs + 3

        def pair_body(jg, _):
            for off in range(2):
                jj_step(2 + jg * 2 + off, off, first=False, launch_next=True)
            return 0

        lax.fori_loop(0, pairs, pair_body, 0)

        jj_step(blocks_ceil - 3, (blocks_ceil - 3) % 2, first=False,
                launch_next=True)
        jj_step(blocks_ceil - 2, (blocks_ceil - 2) % 2, first=False,
                launch_next=False)
        jj_step(blocks_ceil - 1, (blocks_ceil - 1) % 2, first=False,
                launch_next=False)
        wait_out((blocks_ceil - 2) % 2)
        wait_out((blocks_ceil - 1) % 2)

        pltpu.sync_copy(wtail_hbm, tail_v)

        def tail_row(r, _):
            pltpu.sync_copy(
                tail_v.at[r],
                out_hbm.at[pl.ds((NFULL * TBLK + r) * EMBED_DIM, EMBED_DIM)])
            return 0

        lax.fori_loop(0, NTAIL, tail_row, 0)

    return k0


def _make_lookup_kernel(batch, seq):
    total = batch * seq
    per_w = total // NUM_WORKERS
    steps = per_w // CHUNK
    assert per_w % CHUNK == 0 and steps > NBUF

    mesh = plsc.VectorSubcoreMesh(
        core_axis_name="c", subcore_axis_name="s",
        num_cores=NUM_CORES, num_subcores=NUM_SUBCORES)

    @functools.partial(
        pl.kernel,
        out_type=jax.ShapeDtypeStruct((total, EMBED_DIM), jnp.float32),
        mesh=mesh,
        compiler_params=pltpu.CompilerParams(use_tc_tiling_on_sc=False),
        scratch_types=[
            pltpu.VMEM((steps, CHUNK), jnp.int32),
            pltpu.VMEM((2 * SEQ, EMBED_DIM), jnp.float32),
            pltpu.VMEM((NBUF, CHUNK, EMBED_DIM), jnp.float32),
        ]
        + [pltpu.SemaphoreType.DMA] * (2 * NBUF),
    )
    def k1(x_hbm, table_hbm, pe_hbm, out_hbm, idx_v, pe_v, rows_v, *sems):
        sem_g = sems[:NBUF]
        sem_s = sems[NBUF:]
        wid = lax.axis_index("s") * NUM_CORES + lax.axis_index("c")
        base = wid * per_w
        pltpu.sync_copy(x_hbm.at[wid], idx_v)
        pltpu.sync_copy(pe_hbm, pe_v)

        def start_gather(b, kstep):
            pltpu.async_copy(table_hbm.at[idx_v.at[kstep]], rows_v.at[b],
                             sem_g[b])

        def wait_gather(b):
            pltpu.make_async_copy(
                table_hbm.at[pl.ds(0, CHUNK)], rows_v.at[b], sem_g[b]).wait()

        def start_scatter(b, kstep):
            pltpu.async_copy(
                rows_v.at[b], out_hbm.at[pl.ds(base + kstep * CHUNK, CHUNK)],
                sem_s[b])

        def wait_scatter(b):
            pltpu.make_async_copy(
                rows_v.at[b], out_hbm.at[pl.ds(0, CHUNK)], sem_s[b]).wait()

        def add_pe(b, kstep):
            p0 = lax.rem(kstep * CHUNK, seq)

            def add_row(r, _):
                for c in range(EMBED_DIM // LANES):
                    sl = pl.ds(c * LANES, LANES)
                    plsc.addupdate(rows_v.at[b, r, sl], pe_v[p0 + r, sl])
                return 0

            lax.fori_loop(0, CHUNK, add_row, 0, unroll=4)

        def body(b, kstep, relaunch, scatter_wait):
            wait_gather(b)
            add_pe(b, kstep)
            start_scatter(b, kstep)
            if relaunch:
                bn = (b + 2) % NBUF
                if scatter_wait:
                    wait_scatter(bn)
                start_gather(bn, kstep + 2)

        start_gather(0, 0)
        start_gather(1, 1)

        for ks in range(NBUF):
            body(ks, ks, relaunch=True, scatter_wait=ks >= 4)

        groups = (steps - NBUF - 2) // NBUF
        assert steps == NBUF + groups * NBUF + 2

        def loop_body(g, _):
            k0s = NBUF + g * NBUF
            for b in range(NBUF):
                body(b, k0s + b, relaunch=True, scatter_wait=True)
            return 0

        lax.fori_loop(0, groups, loop_body, 0)

        body(0, steps - 2, relaunch=False, scatter_wait=False)
        body(1, steps - 1, relaunch=False, scatter_wait=False)

        for b in range(NBUF):
            wait_scatter(b)

    return k1


def kernel(x, token_embedding_weight):
    batch, seq = x.shape
    total = batch * seq
    w = token_embedding_weight
    k0 = _make_detile_kernel()
    t1d = k0(w.T, w[NFULL * TBLK:])
    tbl = t1d.reshape(VOCAB, EMBED_DIM)
    xi = x.astype(jnp.int32).reshape(NUM_WORKERS, total // (NUM_WORKERS * CHUNK), CHUNK)
    pe1 = _pe_table()
    pe = jnp.concatenate([pe1, pe1], axis=0)
    k1 = _make_lookup_kernel(batch, seq)
    out = k1(xi, tbl, pe)
    return out.reshape(batch, seq, EMBED_DIM)


# speedup vs baseline: 6.5636x; 1.6585x over previous
import functools

import jax
import jax.numpy as jnp
from jax import lax
from jax.experimental import pallas as pl
from jax.experimental.pallas import tpu as pltpu
from jax.experimental.pallas import tpu_sc as plsc

EMBED_DIM = 64
SEQ = 200
LANES = 16

NUM_CORES = 2
NUM_SUBCORES = 16
NUM_WORKERS = NUM_CORES * NUM_SUBCORES

CHUNK = 256
NBUF = 4
PE_REP = 3


def _pe_table():
    pos = jnp.arange(SEQ, dtype=jnp.float32)[:, None]
    i = jnp.arange(0, EMBED_DIM, 2, dtype=jnp.float32)
    div = jnp.exp(-(jnp.log(10000.0) * i / EMBED_DIM))
    pe = jnp.zeros((SEQ, EMBED_DIM), dtype=jnp.float32)
    pe = pe.at[:, 0::2].set(jnp.sin(pos * div))
    pe = pe.at[:, 1::2].set(jnp.cos(pos * div))
    return pe


def _make_kernel(batch, seq):
    total = batch * seq
    per_w = total // NUM_WORKERS
    steps = per_w // CHUNK
    assert per_w % CHUNK == 0 and steps > NBUF + 2
    assert (steps - 4) % NBUF == 0

    mesh = plsc.VectorSubcoreMesh(
        core_axis_name="c", subcore_axis_name="s",
        num_cores=NUM_CORES, num_subcores=NUM_SUBCORES)

    @functools.partial(
        pl.kernel,
        out_type=jax.ShapeDtypeStruct((total, EMBED_DIM), jnp.float32),
        mesh=mesh,
        compiler_params=pltpu.CompilerParams(use_tc_tiling_on_sc=False),
        scratch_types=[
            pltpu.VMEM((steps, CHUNK), jnp.int32),
            pltpu.VMEM((PE_REP * SEQ, EMBED_DIM), jnp.float32),
            pltpu.VMEM((NBUF, CHUNK, EMBED_DIM), jnp.float32),
        ]
        + [pltpu.SemaphoreType.DMA] * (2 * NBUF),
    )
    def k(x_hbm, table_hbm, pe_hbm, out_hbm, idx_v, pe_v, rows_v, *sems):
        sem_g = sems[:NBUF]
        sem_s = sems[NBUF:]
        wid = lax.axis_index("s") * NUM_CORES + lax.axis_index("c")
        base = wid * per_w
        pltpu.sync_copy(x_hbm.at[wid], idx_v)
        pltpu.sync_copy(pe_hbm, pe_v)

        def start_gather(b, kstep):
            pltpu.async_copy(table_hbm.at[idx_v.at[kstep]], rows_v.at[b],
                             sem_g[b])

        def wait_gather(b):
            pltpu.make_async_copy(
                table_hbm.at[pl.ds(0, CHUNK)], rows_v.at[b], sem_g[b]).wait()

        def start_scatter(b, kstep):
            pltpu.async_copy(
                rows_v.at[b], out_hbm.at[pl.ds(base + kstep * CHUNK, CHUNK)],
                sem_s[b])

        def wait_scatter(b):
            pltpu.make_async_copy(
                rows_v.at[b], out_hbm.at[pl.ds(0, CHUNK)], sem_s[b]).wait()

        def add_pe(b, kstep):
            p0 = lax.rem(kstep * CHUNK, seq)

            def add_row(r, _):
                for c in range(EMBED_DIM // LANES):
                    sl = pl.ds(c * LANES, LANES)
                    plsc.addupdate(rows_v.at[b, r, sl], pe_v[p0 + r, sl])
                return 0

            lax.fori_loop(0, CHUNK, add_row, 0, unroll=4)

        def body(b, kstep, relaunch, scatter_wait):
            wait_gather(b)
            add_pe(b, kstep)
            start_scatter(b, kstep)
            if relaunch:
                bn = (b + 2) % NBUF
                if scatter_wait:
                    wait_scatter(bn)
                start_gather(bn, kstep + 2)

        start_gather(0, 0)
        start_gather(1, 1)

        for ks in range(2):
            body(ks, ks, relaunch=True, scatter_wait=False)

        groups = (steps - 4) // NBUF

        def loop_body(g, _):
            k0s = 2 + g * NBUF
            for off in range(NBUF):
                body((2 + off) % NBUF, k0s + off, relaunch=True,
                     scatter_wait=True)
            return 0

        lax.fori_loop(0, groups, loop_body, 0)

        body((steps - 2) % NBUF, steps - 2, relaunch=False, scatter_wait=False)
        body((steps - 1) % NBUF, steps - 1, relaunch=False, scatter_wait=False)

        for b in range(NBUF):
            wait_scatter(b)

    return k


def kernel(x, token_embedding_weight):
    batch, seq = x.shape
    total = batch * seq
    xi = x.astype(jnp.int32).reshape(NUM_WORKERS, total // (NUM_WORKERS * CHUNK), CHUNK)
    pe1 = _pe_table()
    pe = jnp.concatenate([pe1] * PE_REP, axis=0)
    k = _make_kernel(batch, seq)
    out = k(xi, token_embedding_weight, pe)
    return out.reshape(batch, seq, EMBED_DIM)


# speedup vs baseline: 6.5659x; 1.0003x over previous
import functools

import jax
import jax.numpy as jnp
from jax import lax
from jax.experimental import pallas as pl
from jax.experimental.pallas import tpu as pltpu
from jax.experimental.pallas import tpu_sc as plsc

EMBED_DIM = 64
SEQ = 200
LANES = 16

NUM_CORES = 2
NUM_SUBCORES = 16
NUM_WORKERS = NUM_CORES * NUM_SUBCORES

CHUNK = 128
NBUF = 6
PE_REP = 2


def _pe_table():
    pos = jnp.arange(SEQ, dtype=jnp.float32)[:, None]
    i = jnp.arange(0, EMBED_DIM, 2, dtype=jnp.float32)
    div = jnp.exp(-(jnp.log(10000.0) * i / EMBED_DIM))
    pe = jnp.zeros((SEQ, EMBED_DIM), dtype=jnp.float32)
    pe = pe.at[:, 0::2].set(jnp.sin(pos * div))
    pe = pe.at[:, 1::2].set(jnp.cos(pos * div))
    return pe


def _make_kernel(batch, seq):
    total = batch * seq
    per_w = total // NUM_WORKERS
    steps = per_w // CHUNK
    assert per_w % CHUNK == 0 and steps > NBUF + 2
    assert (steps - NBUF - 2) % NBUF == 0

    mesh = plsc.VectorSubcoreMesh(
        core_axis_name="c", subcore_axis_name="s",
        num_cores=NUM_CORES, num_subcores=NUM_SUBCORES)

    @functools.partial(
        pl.kernel,
        out_type=jax.ShapeDtypeStruct((total, EMBED_DIM), jnp.float32),
        mesh=mesh,
        compiler_params=pltpu.CompilerParams(use_tc_tiling_on_sc=False),
        scratch_types=[
            pltpu.VMEM((steps, CHUNK), jnp.int32),
            pltpu.VMEM((PE_REP * SEQ, EMBED_DIM), jnp.float32),
            pltpu.VMEM((NBUF, CHUNK, EMBED_DIM), jnp.float32),
        ]
        + [pltpu.SemaphoreType.DMA] * (2 * NBUF),
    )
    def k(x_hbm, table_hbm, pe_hbm, out_hbm, idx_v, pe_v, rows_v, *sems):
        sem_g = sems[:NBUF]
        sem_s = sems[NBUF:]
        wid = lax.axis_index("s") * NUM_CORES + lax.axis_index("c")
        base = wid * per_w
        pltpu.sync_copy(x_hbm.at[wid], idx_v)
        pltpu.sync_copy(pe_hbm, pe_v)

        def start_gather(b, kstep):
            pltpu.async_copy(table_hbm.at[idx_v.at[kstep]], rows_v.at[b],
                             sem_g[b])

        def wait_gather(b):
            pltpu.make_async_copy(
                table_hbm.at[pl.ds(0, CHUNK)], rows_v.at[b], sem_g[b]).wait()

        def start_scatter(b, kstep):
            pltpu.async_copy(
                rows_v.at[b], out_hbm.at[pl.ds(base + kstep * CHUNK, CHUNK)],
                sem_s[b])

        def wait_scatter(b):
            pltpu.make_async_copy(
                rows_v.at[b], out_hbm.at[pl.ds(0, CHUNK)], sem_s[b]).wait()

        def add_pe(b, kstep):
            p0 = lax.rem(kstep * CHUNK, seq)

            def add_row(r, _):
                for c in range(EMBED_DIM // LANES):
                    sl = pl.ds(c * LANES, LANES)
                    plsc.addupdate(rows_v.at[b, r, sl], pe_v[p0 + r, sl])
                return 0

            lax.fori_loop(0, CHUNK, add_row, 0, unroll=4)

        def body(b, kstep, relaunch, scatter_wait):
            wait_gather(b)
            add_pe(b, kstep)
            start_scatter(b, kstep)
            if relaunch:
                bn = (b + 2) % NBUF
                if scatter_wait:
                    wait_scatter(bn)
                start_gather(bn, kstep + 2)

        start_gather(0, 0)
        start_gather(1, 1)

        for ks in range(NBUF):
            body(ks, ks, relaunch=True, scatter_wait=ks >= 4)

        groups = (steps - NBUF - 2) // NBUF

        def loop_body(g, _):
            k0s = NBUF + g * NBUF
            for off in range(NBUF):
                body(off, k0s + off, relaunch=True, scatter_wait=True)
            return 0

        lax.fori_loop(0, groups, loop_body, 0)

        body((steps - 2) % NBUF, steps - 2, relaunch=False, scatter_wait=False)
        body((steps - 1) % NBUF, steps - 1, relaunch=False, scatter_wait=False)

        for b in range(NBUF):
            wait_scatter(b)

    return k


def kernel(x, token_embedding_weight):
    batch, seq = x.shape
    total = batch * seq
    xi = x.astype(jnp.int32).reshape(NUM_WORKERS, total // (NUM_WORKERS * CHUNK), CHUNK)
    pe1 = _pe_table()
    pe = jnp.concatenate([pe1] * PE_REP, axis=0)
    k = _make_kernel(batch, seq)
    out = k(xi, token_embedding_weight, pe)
    return out.reshape(batch, seq, EMBED_DIM)
